# baseline (device time: 35945 ns/iter reference)
import functools

import jax
import jax.numpy as jnp
from jax import lax
from jax.experimental import pallas as pl
from jax.experimental.pallas import tpu as pltpu

N_DEV = 8
B = 128
D = 128
ROUNDS = (1, 2, 4)
N_ROUNDS = 9


def kernel(x, Win0, Wout0, Win1, Wout1, Win2, Wout2):
    def body(x_ref, win0_ref, wout0_ref, win1_ref, wout1_ref,
             win2_ref, wout2_ref, out_ref,
             send_buf, recv_buf, send_sems, recv_sems):
        my = lax.axis_index("i")
        partners = [my ^ d for d in ROUNDS]

        barrier_sem = pltpu.get_barrier_semaphore()
        for p in partners:
            pl.semaphore_signal(
                barrier_sem, inc=1,
                device_id=(p,), device_id_type=pl.DeviceIdType.MESH,
            )
        pl.semaphore_wait(barrier_sem, 3)

        def all_reduce(p_val, layer):
            for r, d in enumerate(ROUNDS):
                slot = layer * 3 + r
                partner = my ^ d
                send_buf[slot] = p_val
                rdma = pltpu.make_async_remote_copy(
                    src_ref=send_buf.at[slot],
                    dst_ref=recv_buf.at[slot],
                    send_sem=send_sems.at[slot],
                    recv_sem=recv_sems.at[slot],
                    device_id=(partner,),
                    device_id_type=pl.DeviceIdType.MESH,
                )
                rdma.start()
                rdma.wait()
                p_val = p_val + recv_buf[slot]
            return p_val

        xv = x_ref[:].astype(jnp.bfloat16)
        for layer, (win, wout) in enumerate([
            (win0_ref, wout0_ref),
            (win1_ref, wout1_ref),
            (win2_ref, wout2_ref),
        ]):
            h = jnp.dot(xv, win[:].astype(jnp.bfloat16),
                        preferred_element_type=jnp.float32)
            h = jnp.maximum(h, 0.0).astype(jnp.bfloat16)
            partial = jnp.dot(h, wout[:].astype(jnp.bfloat16),
                              preferred_element_type=jnp.float32)
            reduced = all_reduce(partial, layer)
            xv = reduced.astype(jnp.bfloat16)

        out_ref[:] = reduced

        @functools.partial(
            pl.run_scoped, exit_sem=pltpu.SemaphoreType.REGULAR)
        def _(exit_sem):
            for p in partners:
                pl.semaphore_signal(
                    exit_sem, inc=1,
                    device_id=(p,), device_id_type=pl.DeviceIdType.MESH,
                )
            pl.semaphore_wait(exit_sem, 3)

    return pl.pallas_call(
        body,
        out_shape=jax.ShapeDtypeStruct((B, D), jnp.float32),
        in_specs=[pl.BlockSpec(memory_space=pltpu.VMEM)] * 7,
        out_specs=pl.BlockSpec(memory_space=pltpu.VMEM),
        scratch_shapes=[
            pltpu.VMEM((N_ROUNDS, B, D), jnp.float32),
            pltpu.VMEM((N_ROUNDS, B, D), jnp.float32),
            pltpu.SemaphoreType.DMA((N_ROUNDS,)),
            pltpu.SemaphoreType.DMA((N_ROUNDS,)),
        ],
        compiler_params=pltpu.CompilerParams(collective_id=0),
    )(x, Win0, Wout0, Win1, Wout1, Win2, Wout2)


# device time: 32756 ns/iter; 1.0974x vs baseline; 1.0974x over previous
import functools

import jax
import jax.numpy as jnp
from jax import lax
from jax.experimental import pallas as pl
from jax.experimental.pallas import tpu as pltpu

N_DEV = 8
B = 128
D = 128
ROUNDS = (1, 2, 4)
N_ROUNDS = 9


def kernel(x, Win0, Wout0, Win1, Wout1, Win2, Wout2):
    def body(x_ref, win0_ref, wout0_ref, win1_ref, wout1_ref,
             win2_ref, wout2_ref, out_ref,
             send_buf, recv_buf, send_sems, recv_sems):
        my = lax.axis_index("i")
        partners = [my ^ d for d in ROUNDS]

        barrier_sem = pltpu.get_barrier_semaphore()
        for p in partners:
            pl.semaphore_signal(
                barrier_sem, inc=1,
                device_id=(p,), device_id_type=pl.DeviceIdType.MESH,
            )
        pl.semaphore_wait(barrier_sem, 3)

        def all_reduce(p_val, layer):
            for r, d in enumerate(ROUNDS):
                slot = layer * 3 + r
                partner = my ^ d
                send_buf[slot] = p_val.astype(jnp.bfloat16)
                rdma = pltpu.make_async_remote_copy(
                    src_ref=send_buf.at[slot],
                    dst_ref=recv_buf.at[slot],
                    send_sem=send_sems.at[slot],
                    recv_sem=recv_sems.at[slot],
                    device_id=(partner,),
                    device_id_type=pl.DeviceIdType.MESH,
                )
                rdma.start()
                rdma.wait()
                p_val = p_val + recv_buf[slot].astype(jnp.float32)
            return p_val

        xv = x_ref[:].astype(jnp.bfloat16)
        for layer, (win, wout) in enumerate([
            (win0_ref, wout0_ref),
            (win1_ref, wout1_ref),
            (win2_ref, wout2_ref),
        ]):
            h = jnp.dot(xv, win[:].astype(jnp.bfloat16),
                        preferred_element_type=jnp.float32)
            h = jnp.maximum(h, 0.0).astype(jnp.bfloat16)
            partial = jnp.dot(h, wout[:].astype(jnp.bfloat16),
                              preferred_element_type=jnp.float32)
            reduced = all_reduce(partial, layer)
            xv = reduced.astype(jnp.bfloat16)

        out_ref[:] = reduced

        @functools.partial(
            pl.run_scoped, exit_sem=pltpu.SemaphoreType.REGULAR)
        def _(exit_sem):
            for p in partners:
                pl.semaphore_signal(
                    exit_sem, inc=1,
                    device_id=(p,), device_id_type=pl.DeviceIdType.MESH,
                )
            pl.semaphore_wait(exit_sem, 3)

    return pl.pallas_call(
        body,
        out_shape=jax.ShapeDtypeStruct((B, D), jnp.float32),
        in_specs=[pl.BlockSpec(memory_space=pltpu.VMEM)] * 7,
        out_specs=pl.BlockSpec(memory_space=pltpu.VMEM),
        scratch_shapes=[
            pltpu.VMEM((N_ROUNDS, B, D), jnp.bfloat16),
            pltpu.VMEM((N_ROUNDS, B, D), jnp.bfloat16),
            pltpu.SemaphoreType.DMA((N_ROUNDS,)),
            pltpu.SemaphoreType.DMA((N_ROUNDS,)),
        ],
        compiler_params=pltpu.CompilerParams(collective_id=0),
    )(x, Win0, Wout0, Win1, Wout1, Win2, Wout2)


# device time: 26892 ns/iter; 1.3366x vs baseline; 1.2181x over previous
import functools

import jax
import jax.numpy as jnp
from jax import lax
from jax.experimental import pallas as pl
from jax.experimental.pallas import tpu as pltpu

N_DEV = 8
B = 128
D = 128
N_PEERS = N_DEV - 1
N_SLOTS = 3 * N_PEERS


def kernel(x, Win0, Wout0, Win1, Wout1, Win2, Wout2):
    def body(x_ref, win0_ref, wout0_ref, win1_ref, wout1_ref,
             win2_ref, wout2_ref, out_ref,
             send_buf, recv_buf, send_sems, recv_sems):
        my = lax.axis_index("i")
        peers = [my ^ k for k in range(1, N_DEV)]

        barrier_sem = pltpu.get_barrier_semaphore()
        for p in peers:
            pl.semaphore_signal(
                barrier_sem, inc=1,
                device_id=(p,), device_id_type=pl.DeviceIdType.MESH,
            )
        pl.semaphore_wait(barrier_sem, N_PEERS)

        def all_reduce(p_val, layer):
            send_buf[layer] = p_val.astype(jnp.bfloat16)
            rdmas = []
            for k in range(1, N_DEV):
                slot = layer * N_PEERS + (k - 1)
                rdma = pltpu.make_async_remote_copy(
                    src_ref=send_buf.at[layer],
                    dst_ref=recv_buf.at[slot],
                    send_sem=send_sems.at[slot],
                    recv_sem=recv_sems.at[slot],
                    device_id=(my ^ k,),
                    device_id_type=pl.DeviceIdType.MESH,
                )
                rdma.start()
                rdmas.append(rdma)
            for k, rdma in zip(range(1, N_DEV), rdmas):
                slot = layer * N_PEERS + (k - 1)
                rdma.wait()
                p_val = p_val + recv_buf[slot].astype(jnp.float32)
            return p_val

        xv = x_ref[:].astype(jnp.bfloat16)
        for layer, (win, wout) in enumerate([
            (win0_ref, wout0_ref),
            (win1_ref, wout1_ref),
            (win2_ref, wout2_ref),
        ]):
            h = jnp.dot(xv, win[:].astype(jnp.bfloat16),
                        preferred_element_type=jnp.float32)
            h = jnp.maximum(h, 0.0).astype(jnp.bfloat16)
            partial = jnp.dot(h, wout[:].astype(jnp.bfloat16),
                              preferred_element_type=jnp.float32)
            reduced = all_reduce(partial, layer)
            xv = reduced.astype(jnp.bfloat16)

        out_ref[:] = reduced

        @functools.partial(
            pl.run_scoped, exit_sem=pltpu.SemaphoreType.REGULAR)
        def _(exit_sem):
            for p in peers:
                pl.semaphore_signal(
                    exit_sem, inc=1,
                    device_id=(p,), device_id_type=pl.DeviceIdType.MESH,
                )
            pl.semaphore_wait(exit_sem, N_PEERS)

    return pl.pallas_call(
        body,
        out_shape=jax.ShapeDtypeStruct((B, D), jnp.float32),
        in_specs=[pl.BlockSpec(memory_space=pltpu.VMEM)] * 7,
        out_specs=pl.BlockSpec(memory_space=pltpu.VMEM),
        scratch_shapes=[
            pltpu.VMEM((3, B, D), jnp.bfloat16),
            pltpu.VMEM((N_SLOTS, B, D), jnp.bfloat16),
            pltpu.SemaphoreType.DMA((N_SLOTS,)),
            pltpu.SemaphoreType.DMA((N_SLOTS,)),
        ],
        compiler_params=pltpu.CompilerParams(collective_id=0),
    )(x, Win0, Wout0, Win1, Wout1, Win2, Wout2)


# device time: 23201 ns/iter; 1.5493x vs baseline; 1.1591x over previous
import jax
import jax.numpy as jnp
from jax import lax
from jax.experimental import pallas as pl
from jax.experimental.pallas import tpu as pltpu

N_DEV = 8
B = 128
D = 128
H = 256
N_PEERS = N_DEV - 1
N_SLOTS = 3 * N_PEERS
HOP_ORDER = (1, 3, 4, 2, 5, 7, 6)
NEAR_ORDER = HOP_ORDER[:-1]
FAR = 6


def kernel(x, Win0, Wout0, Win1, Wout1, Win2, Wout2):
    def body(x_ref, w_ref, out_ref,
             send_buf, recv_buf, send_sems, recv_sems, ready_sems,
             out_vmem, out_sem):
        my = lax.axis_index("i")

        barrier_sem = pltpu.get_barrier_semaphore()
        pl.semaphore_signal(
            barrier_sem, inc=1,
            device_id=(my,), device_id_type=pl.DeviceIdType.MESH,
        )
        pl.semaphore_wait(barrier_sem, 1)

        for k in range(1, N_DEV):
            pl.semaphore_signal(
                ready_sems.at[k - 1], inc=1,
                device_id=(my ^ k,), device_id_type=pl.DeviceIdType.MESH,
            )

        xv = x_ref[:]

        def win_of(layer):
            return w_ref[layer * B:(layer + 1) * B, :]

        def woutT_of(layer):
            return w_ref[(3 + layer) * B:(4 + layer) * B, :]

        def start_ar(p_val, layer):
            send_buf[layer] = p_val.astype(jnp.bfloat16)
            rdmas = {}
            for k in HOP_ORDER:
                slot = layer * N_PEERS + (k - 1)
                rdma = pltpu.make_async_remote_copy(
                    src_ref=send_buf.at[layer],
                    dst_ref=recv_buf.at[slot],
                    send_sem=send_sems.at[slot],
                    recv_sem=recv_sems.at[slot],
                    device_id=(my ^ k,),
                    device_id_type=pl.DeviceIdType.MESH,
                )
                if layer == 0:
                    pl.semaphore_wait(ready_sems.at[k - 1], 1)
                rdma.start()
                rdmas[k] = rdma
            return rdmas

        def drain(rdmas, layer, ks, acc):
            for k in ks:
                slot = layer * N_PEERS + (k - 1)
                rdmas[k].wait_recv()
                acc = acc + recv_buf[slot].astype(jnp.float32)
            return acc

        h = jnp.dot(xv, win_of(0), preferred_element_type=jnp.float32)
        h = jnp.maximum(h, 0.0).astype(jnp.bfloat16)
        partial = lax.dot_general(
            h, woutT_of(0), (((1,), (1,)), ((), ())),
            preferred_element_type=jnp.float32,
        )
        rdmas = start_ar(partial, 0)

        for layer in range(2):
            near = drain(rdmas, layer, NEAR_ORDER, partial)
            m = jnp.dot(near.astype(jnp.bfloat16), win_of(layer + 1),
                        preferred_element_type=jnp.float32)
            far_slot = layer * N_PEERS + (FAR - 1)
            rdmas[FAR].wait_recv()
            mf = jnp.dot(recv_buf[far_slot], win_of(layer + 1),
                         preferred_element_type=jnp.float32)
            h = jnp.maximum(m + mf, 0.0).astype(jnp.bfloat16)
            partial = lax.dot_general(
                h, woutT_of(layer + 1), (((1,), (1,)), ((), ())),
                preferred_element_type=jnp.float32,
            )
            for k in HOP_ORDER:
                rdmas[k].wait_send()
            rdmas = start_ar(partial, layer + 1)

        total = drain(rdmas, 2, HOP_ORDER, partial)
        for k in HOP_ORDER:
            rdmas[k].wait_send()

        out_vmem[:] = total
        out_dma = pltpu.make_async_copy(out_vmem, out_ref, out_sem)
        out_dma.start()
        out_dma.wait()

    bf = jnp.bfloat16
    w_all = jnp.concatenate(
        [
            Win0.astype(bf), Win1.astype(bf), Win2.astype(bf),
            Wout0.astype(bf).T, Wout1.astype(bf).T, Wout2.astype(bf).T,
        ],
        axis=0,
    )

    return pl.pallas_call(
        body,
        out_shape=jax.ShapeDtypeStruct((B, D), jnp.float32),
        in_specs=[pl.BlockSpec(memory_space=pltpu.VMEM)] * 2,
        out_specs=pl.BlockSpec(memory_space=pltpu.MemorySpace.HBM),
        scratch_shapes=[
            pltpu.VMEM((3, B, D), jnp.bfloat16),
            pltpu.VMEM((N_SLOTS, B, D), jnp.bfloat16),
            pltpu.SemaphoreType.DMA((N_SLOTS,)),
            pltpu.SemaphoreType.DMA((N_SLOTS,)),
            pltpu.SemaphoreType.REGULAR((N_PEERS,)),
            pltpu.VMEM((B, D), jnp.float32),
            pltpu.SemaphoreType.DMA,
        ],
        compiler_params=pltpu.CompilerParams(collective_id=0),
    )(x.astype(bf), w_all)
